# Initial kernel scaffold; baseline (speedup 1.0000x reference)
import jax
import jax.numpy as jnp
from jax.experimental import pallas as pl


def kernel(x1_nchw, x2_nchw, w_up, b_up, w_conv, b_conv, gamma, beta):
    raise NotImplementedError("write your pallas kernel here")


# speedup vs baseline: 2.7831x; 2.7831x over previous
import jax
import jax.numpy as jnp
from jax import lax
from jax.experimental import pallas as pl
from jax.experimental.pallas import tpu as pltpu


def _convt_kernel(x_ref, w_ref, b_ref, o_ref):
    o_ref[0] = (jnp.dot(w_ref[...], x_ref[0].astype(jnp.bfloat16),
                        preferred_element_type=jnp.float32)
                + b_ref[...])


def _make_conv_bn_kernel(W, HW):
    def _conv_bn_kernel(x2_ref, up_ref, w_ref, b_ref, o_ref, ps_ref, pq_ref):
        xc = jnp.concatenate([x2_ref[0], up_ref[0]],
                             axis=0).astype(jnp.bfloat16)
        cin = xc.shape[0]
        z = jnp.zeros((cin, W), jnp.bfloat16)
        xm = jnp.concatenate([z, xc, z], axis=1)
        L = xm.shape[1]
        lane = lax.broadcasted_iota(jnp.int32, (1, L), 1) % W
        zc = jnp.zeros((cin, 1), jnp.bfloat16)
        xl = jnp.where(lane == 0, jnp.bfloat16(0),
                       jnp.concatenate([zc, xm[:, :-1]], axis=1))
        xr = jnp.where(lane == W - 1, jnp.bfloat16(0),
                       jnp.concatenate([xm[:, 1:], zc], axis=1))
        x3 = jnp.concatenate([xl, xm, xr], axis=0)
        out = b_ref[...].astype(jnp.float32)
        for dy in range(3):
            out = out + jnp.dot(w_ref[dy], x3[:, dy * W:dy * W + HW],
                                preferred_element_type=jnp.float32)
        o_ref[0] = out
        ps_ref[0] = jnp.sum(out, axis=1, keepdims=True)
        pq_ref[0] = jnp.sum(out * out, axis=1, keepdims=True)

    return _conv_bn_kernel


def _make_bn_relu_kernel(count, eps):
    def _bn_relu_kernel(x_ref, ps_ref, pq_ref, g_ref, bt_ref, o_ref):
        s1 = jnp.sum(ps_ref[...], axis=0)
        s2 = jnp.sum(pq_ref[...], axis=0)
        inv = 1.0 / count
        mean = s1 * inv
        var = s2 * inv - mean * mean
        scale = g_ref[...] * lax.rsqrt(var + eps)
        shift = bt_ref[...] - mean * scale
        o_ref[0] = jnp.maximum(x_ref[0] * scale + shift, 0.0)

    return _bn_relu_kernel


def kernel(x1_nchw, x2_nchw, w_up, b_up, w_conv, b_conv, gamma, beta):
    N, C1, H1, W1 = x1_nchw.shape
    _, C2, H, W = x2_nchw.shape
    Cup = w_up.shape[1]
    Cout = w_conv.shape[0]
    Cin = C2 + Cup
    P1 = H1 * W1
    HW = H * W
    f32 = jnp.float32

    x1r = x1_nchw.reshape(N, C1, P1)
    wup = w_up.transpose(2, 3, 1, 0).reshape(4 * Cup, C1).astype(jnp.bfloat16)
    b4 = jnp.tile(b_up, 4).reshape(4 * Cup, 1).astype(f32)
    out4 = pl.pallas_call(
        _convt_kernel,
        out_shape=jax.ShapeDtypeStruct((N, 4 * Cup, P1), f32),
        grid=(N,),
        in_specs=[pl.BlockSpec((1, C1, P1), lambda n: (n, 0, 0)),
                  pl.BlockSpec((4 * Cup, C1), lambda n: (0, 0)),
                  pl.BlockSpec((4 * Cup, 1), lambda n: (0, 0))],
        out_specs=pl.BlockSpec((1, 4 * Cup, P1), lambda n: (n, 0, 0)),
        compiler_params=pltpu.CompilerParams(
            dimension_semantics=("parallel",)),
    )(x1r, wup, b4)
    up = (out4.reshape(N, 2, 2, Cup, H1, W1)
          .transpose(0, 3, 4, 1, 5, 2)
          .reshape(N, Cup, HW))

    x2r = x2_nchw.reshape(N, C2, HW)
    wt = (w_conv.transpose(2, 0, 3, 1)
          .reshape(3, Cout, 3 * Cin).astype(jnp.bfloat16))
    bc = b_conv.reshape(Cout, 1).astype(f32)
    conv, psum, pssq = pl.pallas_call(
        _make_conv_bn_kernel(W, HW),
        out_shape=(jax.ShapeDtypeStruct((N, Cout, HW), f32),
                   jax.ShapeDtypeStruct((N, Cout, 1), f32),
                   jax.ShapeDtypeStruct((N, Cout, 1), f32)),
        grid=(N,),
        in_specs=[pl.BlockSpec((1, C2, HW), lambda n: (n, 0, 0)),
                  pl.BlockSpec((1, Cup, HW), lambda n: (n, 0, 0)),
                  pl.BlockSpec((3, Cout, 3 * Cin), lambda n: (0, 0, 0)),
                  pl.BlockSpec((Cout, 1), lambda n: (0, 0))],
        out_specs=(pl.BlockSpec((1, Cout, HW), lambda n: (n, 0, 0)),
                   pl.BlockSpec((1, Cout, 1), lambda n: (n, 0, 0)),
                   pl.BlockSpec((1, Cout, 1), lambda n: (n, 0, 0))),
        compiler_params=pltpu.CompilerParams(
            dimension_semantics=("parallel",)),
    )(x2r, up, wt, bc)

    y = pl.pallas_call(
        _make_bn_relu_kernel(float(N * HW), 1e-5),
        out_shape=jax.ShapeDtypeStruct((N, Cout, HW), f32),
        grid=(N,),
        in_specs=[pl.BlockSpec((1, Cout, HW), lambda n: (n, 0, 0)),
                  pl.BlockSpec((N, Cout, 1), lambda n: (0, 0, 0)),
                  pl.BlockSpec((N, Cout, 1), lambda n: (0, 0, 0)),
                  pl.BlockSpec((Cout, 1), lambda n: (0, 0)),
                  pl.BlockSpec((Cout, 1), lambda n: (0, 0))],
        out_specs=pl.BlockSpec((1, Cout, HW), lambda n: (n, 0, 0)),
        compiler_params=pltpu.CompilerParams(
            dimension_semantics=("parallel",)),
    )(conv, psum, pssq, gamma.reshape(Cout, 1).astype(f32),
      beta.reshape(Cout, 1).astype(f32))

    return y.reshape(N, Cout, H, W)


# speedup vs baseline: 3.4948x; 1.2557x over previous
import jax
import jax.numpy as jnp
from jax import lax
from jax.experimental import pallas as pl
from jax.experimental.pallas import tpu as pltpu


def _shift_dn(a, s):
    return jnp.concatenate([a[:, s:], jnp.zeros((a.shape[0], s), a.dtype)],
                           axis=1)


def _shift_up(a, s):
    return jnp.concatenate([jnp.zeros((a.shape[0], s), a.dtype), a[:, :-s]],
                           axis=1)


def _make_fused_kernel(H1, W1, W, HW):
    hw = W // 2

    def _fused_kernel(x1_ref, x2_ref, wu_ref, bu_ref, w_ref, b_ref,
                      o_ref, ps_ref, pq_ref, up_ref):
        t = ((jnp.dot(wu_ref[...], x1_ref[0].astype(jnp.bfloat16),
                      preferred_element_type=jnp.float32)
              + bu_ref[...]).astype(jnp.bfloat16))
        cup = up_ref.shape[0]
        for i in range(H1):
            src = t[:, i * W1:(i + 1) * W1]
            for di in range(2):
                for dj in range(2):
                    k = 2 * di + dj
                    up_ref[:, pl.ds((2 * i + di) * W + hw * dj, W1)] = (
                        src[k * cup:(k + 1) * cup, :])
            del src
        xc = jnp.concatenate([x2_ref[0].astype(jnp.bfloat16), up_ref[...]],
                             axis=0)
        cin = xc.shape[0]
        z = jnp.zeros((cin, W), jnp.bfloat16)
        xm = jnp.concatenate([z, xc, z], axis=1)
        L = xm.shape[1]
        lane = lax.broadcasted_iota(jnp.int32, (1, L), 1)
        px = (lane % W) // hw
        u = lane % hw
        zero = jnp.bfloat16(0)
        xl = jnp.where(px == 0,
                       jnp.where(u == 0, zero, _shift_dn(xm, hw - 1)),
                       _shift_up(xm, hw))
        xr = jnp.where(px == 0,
                       _shift_dn(xm, hw),
                       jnp.where(u == hw - 1, zero, _shift_up(xm, hw - 1)))
        x3 = jnp.concatenate([xl, xm, xr], axis=0)
        out = b_ref[...].astype(jnp.float32)
        for dy in range(3):
            out = out + jnp.dot(w_ref[dy], x3[:, dy * W:dy * W + HW],
                                preferred_element_type=jnp.float32)
        o_ref[0] = out.astype(jnp.bfloat16)
        ps_ref[0] = jnp.sum(out, axis=1, keepdims=True)
        pq_ref[0] = jnp.sum(out * out, axis=1, keepdims=True)

    return _fused_kernel


def _make_bn_relu_kernel(count, eps):
    def _bn_relu_kernel(x_ref, ps_ref, pq_ref, g_ref, bt_ref, o_ref):
        s1 = jnp.sum(ps_ref[...], axis=0)
        s2 = jnp.sum(pq_ref[...], axis=0)
        inv = 1.0 / count
        mean = s1 * inv
        var = s2 * inv - mean * mean
        scale = g_ref[...] * lax.rsqrt(var + eps)
        shift = bt_ref[...] - mean * scale
        o_ref[0] = jnp.maximum(
            x_ref[0].astype(jnp.float32) * scale + shift, 0.0)

    return _bn_relu_kernel


def kernel(x1_nchw, x2_nchw, w_up, b_up, w_conv, b_conv, gamma, beta):
    N, C1, H1, W1 = x1_nchw.shape
    _, C2, H, W = x2_nchw.shape
    Cup = w_up.shape[1]
    Cout = w_conv.shape[0]
    Cin = C2 + Cup
    P1 = H1 * W1
    HW = H * W
    hw = W // 2
    f32 = jnp.float32
    bf16 = jnp.bfloat16

    x1r = x1_nchw.reshape(N, C1, P1)
    x2r = (x2_nchw.reshape(N, C2, H, hw, 2)
           .transpose(0, 1, 2, 4, 3).reshape(N, C2, HW))
    wup = w_up.transpose(2, 3, 1, 0).reshape(4 * Cup, C1).astype(bf16)
    b4 = jnp.tile(b_up, 4).reshape(4 * Cup, 1).astype(f32)
    wt = (w_conv.transpose(2, 0, 3, 1)
          .reshape(3, Cout, 3 * Cin).astype(bf16))
    bc = b_conv.reshape(Cout, 1).astype(f32)

    conv, psum, pssq = pl.pallas_call(
        _make_fused_kernel(H1, W1, W, HW),
        out_shape=(jax.ShapeDtypeStruct((N, Cout, HW), bf16),
                   jax.ShapeDtypeStruct((N, Cout, 1), f32),
                   jax.ShapeDtypeStruct((N, Cout, 1), f32)),
        grid=(N,),
        in_specs=[pl.BlockSpec((1, C1, P1), lambda n: (n, 0, 0)),
                  pl.BlockSpec((1, C2, HW), lambda n: (n, 0, 0)),
                  pl.BlockSpec((4 * Cup, C1), lambda n: (0, 0)),
                  pl.BlockSpec((4 * Cup, 1), lambda n: (0, 0)),
                  pl.BlockSpec((3, Cout, 3 * Cin), lambda n: (0, 0, 0)),
                  pl.BlockSpec((Cout, 1), lambda n: (0, 0))],
        out_specs=(pl.BlockSpec((1, Cout, HW), lambda n: (n, 0, 0)),
                   pl.BlockSpec((1, Cout, 1), lambda n: (n, 0, 0)),
                   pl.BlockSpec((1, Cout, 1), lambda n: (n, 0, 0))),
        scratch_shapes=[pltpu.VMEM((Cup, HW), bf16)],
        compiler_params=pltpu.CompilerParams(
            dimension_semantics=("parallel",)),
    )(x1r, x2r, wup, b4, wt, bc)

    y = pl.pallas_call(
        _make_bn_relu_kernel(float(N * HW), 1e-5),
        out_shape=jax.ShapeDtypeStruct((N, Cout, HW), f32),
        grid=(N,),
        in_specs=[pl.BlockSpec((1, Cout, HW), lambda n: (n, 0, 0)),
                  pl.BlockSpec((N, Cout, 1), lambda n: (0, 0, 0)),
                  pl.BlockSpec((N, Cout, 1), lambda n: (0, 0, 0)),
                  pl.BlockSpec((Cout, 1), lambda n: (0, 0)),
                  pl.BlockSpec((Cout, 1), lambda n: (0, 0))],
        out_specs=pl.BlockSpec((1, Cout, HW), lambda n: (n, 0, 0)),
        compiler_params=pltpu.CompilerParams(
            dimension_semantics=("parallel",)),
    )(conv, psum, pssq, gamma.reshape(Cout, 1).astype(f32),
      beta.reshape(Cout, 1).astype(f32))

    return (y.reshape(N, Cout, H, 2, hw)
            .transpose(0, 1, 2, 4, 3).reshape(N, Cout, H, W))


# speedup vs baseline: 4.4133x; 1.2628x over previous
import jax
import jax.numpy as jnp
from jax import lax
from jax.experimental import pallas as pl
from jax.experimental.pallas import tpu as pltpu


def _make_fused_kernel(H1, W1, C2, W, HW):
    W2 = 2 * W

    def _fused_kernel(x1_ref, x2_ref, wu_ref, bu_ref, w_ref, b_ref,
                      o_ref, ps_ref, pq_ref, x3_ref):
        cup = wu_ref.shape[0] // 4
        H = x2_ref.shape[2]
        cin = C2 + cup
        bf = jnp.bfloat16
        x3_ref[cin:2 * cin, pl.ds(0, W2)] = jnp.zeros((cin, W2), bf)
        x3_ref[cin:2 * cin, pl.ds((H + 2) * W, W2)] = (
            jnp.zeros((cin, W2), bf))
        for p in range(H // 2):
            y = 2 * p
            x3_ref[cin:cin + C2, pl.ds((y + 2) * W, W2)] = jnp.concatenate(
                [x2_ref[0, :, y, :], x2_ref[0, :, y + 1, :]],
                axis=1).astype(bf)
        t_all = (jnp.dot(wu_ref[...], x1_ref[0].astype(bf),
                         preferred_element_type=jnp.float32)
                 + bu_ref[...])
        l = jnp.arange(W2)
        idx = jnp.broadcast_to(
            (W1 * (2 * (l // W) + (l % W) % 2) + (l % W) // 2)[None, :],
            (cup, W2))
        for r in range(H1):
            t = t_all[:, r * W1:(r + 1) * W1]
            quad = jnp.concatenate(
                [t[0:cup], t[cup:2 * cup], t[2 * cup:3 * cup],
                 t[3 * cup:4 * cup]], axis=1)
            pair = jnp.take_along_axis(quad, idx, axis=1)
            x3_ref[cin + C2:2 * cin, pl.ds((2 * r + 2) * W, W2)] = (
                pair.astype(bf))
            del t, quad, pair
        xm = x3_ref[cin:2 * cin, :]
        L = xm.shape[1]
        lane = lax.broadcasted_iota(jnp.int32, (1, L), 1) % W
        zc = jnp.zeros((cin, 1), bf)
        x3_ref[0:cin, :] = jnp.where(
            lane == 0, jnp.bfloat16(0),
            jnp.concatenate([zc, xm[:, :-1]], axis=1))
        x3_ref[2 * cin:3 * cin, :] = jnp.where(
            lane == W - 1, jnp.bfloat16(0),
            jnp.concatenate([xm[:, 1:], zc], axis=1))
        out = b_ref[...].astype(jnp.float32)
        for dy in range(3):
            out = out + jnp.dot(w_ref[dy],
                                x3_ref[:, (1 + dy) * W:(1 + dy) * W + HW],
                                preferred_element_type=jnp.float32)
        o_ref[0] = out.astype(jnp.bfloat16)
        ps_ref[0] = jnp.sum(out, axis=1, keepdims=True)
        pq_ref[0] = jnp.sum(out * out, axis=1, keepdims=True)

    return _fused_kernel


def _make_bn_relu_kernel(count, eps, H, W):
    def _bn_relu_kernel(x_ref, ps_ref, pq_ref, g_ref, bt_ref, o_ref):
        s1 = jnp.sum(ps_ref[...], axis=0)
        s2 = jnp.sum(pq_ref[...], axis=0)
        inv = 1.0 / count
        mean = s1 * inv
        var = s2 * inv - mean * mean
        scale = g_ref[...] * lax.rsqrt(var + eps)
        shift = bt_ref[...] - mean * scale
        y = jnp.maximum(x_ref[0].astype(jnp.float32) * scale + shift, 0.0)
        for r in range(H):
            o_ref[0, :, r, :] = y[:, r * W:(r + 1) * W]

    return _bn_relu_kernel


def kernel(x1_nchw, x2_nchw, w_up, b_up, w_conv, b_conv, gamma, beta):
    N, C1, H1, W1 = x1_nchw.shape
    _, C2, H, W = x2_nchw.shape
    Cup = w_up.shape[1]
    Cout = w_conv.shape[0]
    Cin = C2 + Cup
    HW = H * W
    f32 = jnp.float32
    bf16 = jnp.bfloat16

    wup = w_up.transpose(2, 3, 1, 0).reshape(4 * Cup, C1).astype(bf16)
    b4 = jnp.tile(b_up, 4).reshape(4 * Cup, 1).astype(f32)
    wt = (w_conv.transpose(2, 0, 3, 1)
          .reshape(3, Cout, 3 * Cin).astype(bf16))
    bc = b_conv.reshape(Cout, 1).astype(f32)

    conv, psum, pssq = pl.pallas_call(
        _make_fused_kernel(H1, W1, C2, W, HW),
        out_shape=(jax.ShapeDtypeStruct((N, Cout, HW), bf16),
                   jax.ShapeDtypeStruct((N, Cout, 1), f32),
                   jax.ShapeDtypeStruct((N, Cout, 1), f32)),
        grid=(N,),
        in_specs=[pl.BlockSpec((1, C1, H1 * W1), lambda n: (n, 0, 0)),
                  pl.BlockSpec((1, C2, H, W), lambda n: (n, 0, 0, 0)),
                  pl.BlockSpec((4 * Cup, C1), lambda n: (0, 0)),
                  pl.BlockSpec((4 * Cup, 1), lambda n: (0, 0)),
                  pl.BlockSpec((3, Cout, 3 * Cin), lambda n: (0, 0, 0)),
                  pl.BlockSpec((Cout, 1), lambda n: (0, 0))],
        out_specs=(pl.BlockSpec((1, Cout, HW), lambda n: (n, 0, 0)),
                   pl.BlockSpec((1, Cout, 1), lambda n: (n, 0, 0)),
                   pl.BlockSpec((1, Cout, 1), lambda n: (n, 0, 0))),
        scratch_shapes=[pltpu.VMEM((3 * Cin, (H + 4) * W), bf16)],
        compiler_params=pltpu.CompilerParams(
            dimension_semantics=("parallel",)),
    )(x1_nchw.reshape(N, C1, H1 * W1), x2_nchw, wup, b4, wt, bc)

    return pl.pallas_call(
        _make_bn_relu_kernel(float(N * HW), 1e-5, H, W),
        out_shape=jax.ShapeDtypeStruct((N, Cout, H, W), f32),
        grid=(N,),
        in_specs=[pl.BlockSpec((1, Cout, HW), lambda n: (n, 0, 0)),
                  pl.BlockSpec((N, Cout, 1), lambda n: (0, 0, 0)),
                  pl.BlockSpec((N, Cout, 1), lambda n: (0, 0, 0)),
                  pl.BlockSpec((Cout, 1), lambda n: (0, 0)),
                  pl.BlockSpec((Cout, 1), lambda n: (0, 0))],
        out_specs=pl.BlockSpec((1, Cout, H, W), lambda n: (n, 0, 0, 0)),
        compiler_params=pltpu.CompilerParams(
            dimension_semantics=("parallel",)),
    )(conv, psum, pssq, gamma.reshape(Cout, 1).astype(f32),
      beta.reshape(Cout, 1).astype(f32))


# speedup vs baseline: 4.6436x; 1.0522x over previous
import jax
import jax.numpy as jnp
from jax import lax
from jax.experimental import pallas as pl
from jax.experimental.pallas import tpu as pltpu


def _make_fused_kernel(H1, W1, C2, W, HW):
    W2 = 2 * W

    def _fused_kernel(x1_ref, x2_ref, wu_ref, bu_ref, w_ref, b_ref,
                      o_ref, ps_ref, pq_ref, x3_ref):
        cup = wu_ref.shape[0] // 4
        H = HW // W
        cin = C2 + cup
        bf = jnp.bfloat16
        x3_ref[cin:2 * cin, pl.ds(0, W2)] = jnp.zeros((cin, W2), bf)
        x3_ref[cin:2 * cin, pl.ds((H + 2) * W, W2)] = (
            jnp.zeros((cin, W2), bf))
        x3_ref[cin:cin + C2, pl.ds(2 * W, HW)] = x2_ref[0].astype(bf)
        t_all = (jnp.dot(wu_ref[...], x1_ref[0].astype(bf),
                         preferred_element_type=jnp.float32)
                 + bu_ref[...])
        l = jnp.arange(W2)
        idx = jnp.broadcast_to(
            (W1 * (2 * (l // W) + (l % W) % 2) + (l % W) // 2)[None, :],
            (cup, W2))
        for r in range(H1):
            t = t_all[:, r * W1:(r + 1) * W1]
            quad = jnp.concatenate(
                [t[0:cup], t[cup:2 * cup], t[2 * cup:3 * cup],
                 t[3 * cup:4 * cup]], axis=1)
            pair = jnp.take_along_axis(quad, idx, axis=1)
            x3_ref[cin + C2:2 * cin, pl.ds((2 * r + 2) * W, W2)] = (
                pair.astype(bf))
            del t, quad, pair
        xm = x3_ref[cin:2 * cin, :]
        L = xm.shape[1]
        lane = lax.broadcasted_iota(jnp.int32, (1, L), 1) % W
        zc = jnp.zeros((cin, 1), bf)
        x3_ref[0:cin, :] = jnp.where(
            lane == 0, jnp.bfloat16(0),
            jnp.concatenate([zc, xm[:, :-1]], axis=1))
        x3_ref[2 * cin:3 * cin, :] = jnp.where(
            lane == W - 1, jnp.bfloat16(0),
            jnp.concatenate([xm[:, 1:], zc], axis=1))
        out = b_ref[...].astype(jnp.float32)
        for dy in range(3):
            out = out + jnp.dot(w_ref[dy],
                                x3_ref[:, (1 + dy) * W:(1 + dy) * W + HW],
                                preferred_element_type=jnp.float32)
        o_ref[0] = out.astype(jnp.bfloat16)
        ps_ref[0] = jnp.sum(out, axis=1, keepdims=True)
        pq_ref[0] = jnp.sum(out * out, axis=1, keepdims=True)

    return _fused_kernel


def _make_bn_relu_kernel(count, eps):
    def _bn_relu_kernel(x_ref, ps_ref, pq_ref, g_ref, bt_ref, o_ref):
        s1 = jnp.sum(ps_ref[...], axis=0)
        s2 = jnp.sum(pq_ref[...], axis=0)
        inv = 1.0 / count
        mean = s1 * inv
        var = s2 * inv - mean * mean
        scale = g_ref[...] * lax.rsqrt(var + eps)
        shift = bt_ref[...] - mean * scale
        o_ref[0] = jnp.maximum(
            x_ref[0].astype(jnp.float32) * scale + shift, 0.0)

    return _bn_relu_kernel


def kernel(x1_nchw, x2_nchw, w_up, b_up, w_conv, b_conv, gamma, beta):
    N, C1, H1, W1 = x1_nchw.shape
    _, C2, H, W = x2_nchw.shape
    Cup = w_up.shape[1]
    Cout = w_conv.shape[0]
    Cin = C2 + Cup
    HW = H * W
    f32 = jnp.float32
    bf16 = jnp.bfloat16

    wup = w_up.transpose(2, 3, 1, 0).reshape(4 * Cup, C1).astype(bf16)
    b4 = jnp.tile(b_up, 4).reshape(4 * Cup, 1).astype(f32)
    wt = (w_conv.transpose(2, 0, 3, 1)
          .reshape(3, Cout, 3 * Cin).astype(bf16))
    bc = b_conv.reshape(Cout, 1).astype(f32)

    conv, psum, pssq = pl.pallas_call(
        _make_fused_kernel(H1, W1, C2, W, HW),
        out_shape=(jax.ShapeDtypeStruct((N, Cout, HW), bf16),
                   jax.ShapeDtypeStruct((N, Cout, 1), f32),
                   jax.ShapeDtypeStruct((N, Cout, 1), f32)),
        grid=(N,),
        in_specs=[pl.BlockSpec((1, C1, H1 * W1), lambda n: (n, 0, 0)),
                  pl.BlockSpec((1, C2, HW), lambda n: (n, 0, 0)),
                  pl.BlockSpec((4 * Cup, C1), lambda n: (0, 0)),
                  pl.BlockSpec((4 * Cup, 1), lambda n: (0, 0)),
                  pl.BlockSpec((3, Cout, 3 * Cin), lambda n: (0, 0, 0)),
                  pl.BlockSpec((Cout, 1), lambda n: (0, 0))],
        out_specs=(pl.BlockSpec((1, Cout, HW), lambda n: (n, 0, 0)),
                   pl.BlockSpec((1, Cout, 1), lambda n: (n, 0, 0)),
                   pl.BlockSpec((1, Cout, 1), lambda n: (n, 0, 0))),
        scratch_shapes=[pltpu.VMEM((3 * Cin, (H + 4) * W), bf16)],
        compiler_params=pltpu.CompilerParams(
            dimension_semantics=("parallel",)),
    )(x1_nchw.reshape(N, C1, H1 * W1), x2_nchw.reshape(N, C2, HW),
      wup, b4, wt, bc)

    y = pl.pallas_call(
        _make_bn_relu_kernel(float(N * HW), 1e-5),
        out_shape=jax.ShapeDtypeStruct((N, Cout, HW), f32),
        grid=(N,),
        in_specs=[pl.BlockSpec((1, Cout, HW), lambda n: (n, 0, 0)),
                  pl.BlockSpec((N, Cout, 1), lambda n: (0, 0, 0)),
                  pl.BlockSpec((N, Cout, 1), lambda n: (0, 0, 0)),
                  pl.BlockSpec((Cout, 1), lambda n: (0, 0)),
                  pl.BlockSpec((Cout, 1), lambda n: (0, 0))],
        out_specs=pl.BlockSpec((1, Cout, HW), lambda n: (n, 0, 0)),
        compiler_params=pltpu.CompilerParams(
            dimension_semantics=("parallel",)),
    )(conv, psum, pssq, gamma.reshape(Cout, 1).astype(f32),
      beta.reshape(Cout, 1).astype(f32))
    return y.reshape(N, Cout, H, W)
